# Initial kernel scaffold; baseline (speedup 1.0000x reference)
import jax
import jax.numpy as jnp
from jax.experimental import pallas as pl


def kernel(input, table):
    raise NotImplementedError("write your pallas kernel here")


# speedup vs baseline: 1.8313x; 1.8313x over previous
import functools

import jax
import jax.numpy as jnp
from jax import lax
from jax.experimental import pallas as pl
from jax.experimental.pallas import tpu as pltpu
from jax.experimental.pallas import tpu_sc as plsc

NUM_CORES = 2
NUM_SUBCORES = 16
NUM_WORKERS = NUM_CORES * NUM_SUBCORES
CHUNK = 512
GATHER = 128


@functools.partial(jax.jit, static_argnums=())
def _embedding_lookup(idx_flat, table):
    total = idx_flat.shape[0]
    dim = table.shape[1]
    b_per_w = total // NUM_WORKERS
    n_chunks = b_per_w // CHUNK
    mesh = plsc.VectorSubcoreMesh(core_axis_name="c", subcore_axis_name="s")

    @functools.partial(
        pl.kernel,
        mesh=mesh,
        out_type=jax.ShapeDtypeStruct((total, dim), jnp.float32),
        scratch_types=[
            pltpu.VMEM((b_per_w,), jnp.int32),
            pltpu.VMEM((CHUNK, dim), jnp.float32),
            pltpu.SemaphoreType.DMA,
        ],
        compiler_params=pltpu.CompilerParams(use_tc_tiling_on_sc=False),
    )
    def emb(idx_hbm, table_hbm, out_hbm, idx_v, rows_v, sem):
        wid = lax.axis_index("s") * NUM_CORES + lax.axis_index("c")
        base = wid * b_per_w
        pltpu.sync_copy(idx_hbm.at[pl.ds(base, b_per_w)], idx_v)

        def chunk_body(g, carry):
            off = g * CHUNK
            copies = []
            for j in range(CHUNK // GATHER):
                cp = pltpu.async_copy(
                    table_hbm.at[idx_v.at[pl.ds(off + j * GATHER, GATHER)]],
                    rows_v.at[pl.ds(j * GATHER, GATHER)],
                    sem,
                )
                copies.append(cp)
            for cp in copies:
                cp.wait()
            pltpu.sync_copy(rows_v, out_hbm.at[pl.ds(base + off, CHUNK)])
            return carry

        lax.fori_loop(0, n_chunks, chunk_body, 0)

    return emb(idx_flat, table)


def kernel(input, table):
    B, L = input.shape
    dim = table.shape[1]
    idx_flat = input.reshape(B * L)
    out = _embedding_lookup(idx_flat, table)
    return out.reshape(B, L, dim)


# speedup vs baseline: 1.8763x; 1.0246x over previous
import functools

import jax
import jax.numpy as jnp
from jax import lax
from jax.experimental import pallas as pl
from jax.experimental.pallas import tpu as pltpu
from jax.experimental.pallas import tpu_sc as plsc

NUM_CORES = 2
NUM_SUBCORES = 16
NUM_WORKERS = NUM_CORES * NUM_SUBCORES
CHUNK = 512
GATHER = 128
N_G = CHUNK // GATHER


def _embedding_lookup(idx_flat, table):
    total = idx_flat.shape[0]
    dim = table.shape[1]
    b_per_w = total // NUM_WORKERS
    n_chunks = b_per_w // CHUNK
    n_pairs = n_chunks // 2
    mesh = plsc.VectorSubcoreMesh(core_axis_name="c", subcore_axis_name="s")

    @functools.partial(
        pl.kernel,
        mesh=mesh,
        out_type=jax.ShapeDtypeStruct((total, dim), jnp.float32),
        scratch_types=[
            pltpu.VMEM((b_per_w,), jnp.int32),
            pltpu.VMEM((CHUNK, dim), jnp.float32),
            pltpu.VMEM((CHUNK, dim), jnp.float32),
            pltpu.SemaphoreType.DMA,
            pltpu.SemaphoreType.DMA,
            pltpu.SemaphoreType.DMA,
            pltpu.SemaphoreType.DMA,
        ],
        compiler_params=pltpu.CompilerParams(use_tc_tiling_on_sc=False),
    )
    def emb(idx_hbm, table_hbm, out_hbm, idx_v, buf0, buf1,
            gsem0, gsem1, wsem0, wsem1):
        wid = lax.axis_index("s") * NUM_CORES + lax.axis_index("c")
        base = wid * b_per_w
        pltpu.sync_copy(idx_hbm.at[pl.ds(base, b_per_w)], idx_v)

        def fire_gathers(chunk_row, buf, sem):
            for j in range(N_G):
                pltpu.async_copy(
                    table_hbm.at[idx_v.at[pl.ds(chunk_row + j * GATHER, GATHER)]],
                    buf.at[pl.ds(j * GATHER, GATHER)],
                    sem,
                )

        def drain_gathers(buf, sem):
            for j in range(N_G):
                pltpu.make_async_copy(
                    table_hbm.at[idx_v.at[pl.ds(j * GATHER, GATHER)]],
                    buf.at[pl.ds(j * GATHER, GATHER)],
                    sem,
                ).wait()

        def fire_write(buf, chunk_row, sem):
            pltpu.async_copy(buf, out_hbm.at[pl.ds(base + chunk_row, CHUNK)], sem)

        def drain_write(buf, sem):
            pltpu.make_async_copy(
                buf, out_hbm.at[pl.ds(base, CHUNK)], sem
            ).wait()

        fire_gathers(0, buf0, gsem0)

        def pair_body(t, carry):
            c1_row = (2 * t + 1) * CHUNK
            c2_row = (2 * t + 2) * CHUNK

            @pl.when(t > 0)
            def _():
                drain_write(buf1, wsem1)

            fire_gathers(c1_row, buf1, gsem1)
            drain_gathers(buf0, gsem0)
            fire_write(buf0, 2 * t * CHUNK, wsem0)
            drain_write(buf0, wsem0)

            @pl.when(2 * t + 2 < n_chunks)
            def _():
                fire_gathers(c2_row, buf0, gsem0)

            drain_gathers(buf1, gsem1)
            fire_write(buf1, c1_row, wsem1)
            return carry

        lax.fori_loop(0, n_pairs, pair_body, 0)
        drain_write(buf1, wsem1)

    return emb(idx_flat, table)


def kernel(input, table):
    B, L = input.shape
    dim = table.shape[1]
    idx_flat = input.reshape(B * L)
    out = _embedding_lookup(idx_flat, table)
    return out.reshape(B, L, dim)


# speedup vs baseline: 1.8769x; 1.0003x over previous
import functools

import jax
import jax.numpy as jnp
from jax import lax
from jax.experimental import pallas as pl
from jax.experimental.pallas import tpu as pltpu
from jax.experimental.pallas import tpu_sc as plsc

NUM_CORES = 2
NUM_SUBCORES = 16
NUM_WORKERS = NUM_CORES * NUM_SUBCORES
CHUNK = 512
GATHER = 256
N_G = CHUNK // GATHER


def _embedding_lookup(idx_flat, table):
    total = idx_flat.shape[0]
    dim = table.shape[1]
    b_per_w = total // NUM_WORKERS
    n_chunks = b_per_w // CHUNK
    n_pairs = n_chunks // 2
    mesh = plsc.VectorSubcoreMesh(core_axis_name="c", subcore_axis_name="s")

    @functools.partial(
        pl.kernel,
        mesh=mesh,
        out_type=jax.ShapeDtypeStruct((total, dim), jnp.float32),
        scratch_types=[
            pltpu.VMEM((b_per_w,), jnp.int32),
            pltpu.VMEM((CHUNK, dim), jnp.float32),
            pltpu.VMEM((CHUNK, dim), jnp.float32),
            pltpu.SemaphoreType.DMA,
            pltpu.SemaphoreType.DMA,
            pltpu.SemaphoreType.DMA,
            pltpu.SemaphoreType.DMA,
        ],
        compiler_params=pltpu.CompilerParams(use_tc_tiling_on_sc=False),
    )
    def emb(idx_hbm, table_hbm, out_hbm, idx_v, buf0, buf1,
            gsem0, gsem1, wsem0, wsem1):
        wid = lax.axis_index("s") * NUM_CORES + lax.axis_index("c")
        base = wid * b_per_w
        pltpu.sync_copy(idx_hbm.at[pl.ds(base, b_per_w)], idx_v)

        def fire_gathers(chunk_row, buf, sem):
            for j in range(N_G):
                pltpu.async_copy(
                    table_hbm.at[idx_v.at[pl.ds(chunk_row + j * GATHER, GATHER)]],
                    buf.at[pl.ds(j * GATHER, GATHER)],
                    sem,
                )

        def drain_gathers(buf, sem):
            for j in range(N_G):
                pltpu.make_async_copy(
                    table_hbm.at[idx_v.at[pl.ds(j * GATHER, GATHER)]],
                    buf.at[pl.ds(j * GATHER, GATHER)],
                    sem,
                ).wait()

        def fire_write(buf, chunk_row, sem):
            pltpu.async_copy(buf, out_hbm.at[pl.ds(base + chunk_row, CHUNK)], sem)

        def drain_write(buf, sem):
            pltpu.make_async_copy(
                buf, out_hbm.at[pl.ds(base, CHUNK)], sem
            ).wait()

        fire_gathers(0, buf0, gsem0)

        def pair_body(t, carry):
            c1_row = (2 * t + 1) * CHUNK
            c2_row = (2 * t + 2) * CHUNK

            @pl.when(t > 0)
            def _():
                drain_write(buf1, wsem1)

            fire_gathers(c1_row, buf1, gsem1)
            drain_gathers(buf0, gsem0)
            fire_write(buf0, 2 * t * CHUNK, wsem0)
            drain_write(buf0, wsem0)

            @pl.when(2 * t + 2 < n_chunks)
            def _():
                fire_gathers(c2_row, buf0, gsem0)

            drain_gathers(buf1, gsem1)
            fire_write(buf1, c1_row, wsem1)
            return carry

        lax.fori_loop(0, n_pairs, pair_body, 0)
        drain_write(buf1, wsem1)

    return emb(idx_flat, table)


def kernel(input, table):
    B, L = input.shape
    dim = table.shape[1]
    idx_flat = input.reshape(B * L)
    out = _embedding_lookup(idx_flat, table)
    return out.reshape(B, L, dim)
